# baseline (speedup 1.0000x reference)
import functools

import jax
import jax.numpy as jnp
from jax import lax
from jax.experimental import pallas as pl
from jax.experimental.pallas import tpu as pltpu
from jax.experimental.pallas import tpu_sc as plsc

_CAT_COUNTS = [4100, 54, 22, 16, 5, 17, 33, 337, 2, 2, 7, 12, 30, 20, 30]
_B = 1024
_T = 50
_D = 50
_DP = 64
_NUMV = 3
_NV = _NUMV + len(_CAT_COUNTS)
_VOCAB = 4100
_R = _NV * _T

_LIMS = [_VOCAB - 1] * _NUMV + [c - 1 for c in _CAT_COUNTS]
_OFFS = []
_tot = 0
for _c in [_VOCAB] * _NUMV + [c + 1 for c in _CAT_COUNTS]:
    _OFFS.append(_tot)
    _tot += _c
_TOT = _tot

_NW = 32
_NB = _B // _NW


def _num_tables_body(w_ref, b_ref, o0, o1, o2):
    rows = lax.broadcasted_iota(jnp.int32, (_VOCAB, _DP), 0).astype(jnp.float32)
    outs = (o0, o1, o2)
    for i in range(_NUMV):
        outs[i][...] = rows * w_ref[i][None, :] + b_ref[i][None, :]


def _make_num_tables(w, b):
    return pl.pallas_call(
        _num_tables_body,
        out_shape=[jax.ShapeDtypeStruct((_VOCAB, _DP), jnp.float32)] * _NUMV,
    )(w, b)


def _sc_body(xf_hbm, tbl_hbm, out_hbm,
             x_v, idx_a, idx_b, buf_a, buf_b,
             gsem_a, gsem_b, wsem_a, wsem_b):
    cid = lax.axis_index("c")
    sid = lax.axis_index("s")
    wid = sid * 2 + cid
    b0 = wid * _NB

    lanes = lax.iota(jnp.int32, 16)

    def compute_idx(bl, idx_v):
        for j in range(_NV):
            for off in (0, 16, 32, 34):
                t = off + lanes
                flat = bl * (_T * _NV) + t * _NV + j
                vals = plsc.load_gather(x_v, [flat])
                iv = jnp.minimum(vals.astype(jnp.int32), _LIMS[j]) + _OFFS[j]
                idx_v[pl.ds(j * _T + off, 16)] = iv

    def per_pair(i, carry):
        b1 = b0 + 2 * i
        pltpu.sync_copy(xf_hbm.at[pl.ds(b1 * (_T * _NV), 2 * _T * _NV)], x_v)
        compute_idx(0, idx_a)

        @pl.when(i > 0)
        def _():
            pltpu.make_async_copy(buf_a, out_hbm.at[b1 - 2], wsem_a).wait()
        ga = pltpu.async_copy(tbl_hbm.at[idx_a], buf_a, gsem_a)

        compute_idx(1, idx_b)

        @pl.when(i > 0)
        def _():
            pltpu.make_async_copy(buf_b, out_hbm.at[b1 - 1], wsem_b).wait()
        ga.wait()
        pltpu.async_copy(buf_a, out_hbm.at[b1], wsem_a)

        gb = pltpu.async_copy(tbl_hbm.at[idx_b], buf_b, gsem_b)
        gb.wait()
        pltpu.async_copy(buf_b, out_hbm.at[b1 + 1], wsem_b)
        return carry

    lax.fori_loop(0, _NB // 2, per_pair, 0)
    pltpu.make_async_copy(buf_a, out_hbm.at[b0], wsem_a).wait()
    pltpu.make_async_copy(buf_b, out_hbm.at[b0], wsem_b).wait()


@functools.partial(jax.jit)
def _run(xf, tbl):
    k = pl.kernel(
        _sc_body,
        out_type=jax.ShapeDtypeStruct((_B, _R, _DP), jnp.float32),
        mesh=plsc.VectorSubcoreMesh(core_axis_name="c", subcore_axis_name="s"),
        compiler_params=pltpu.CompilerParams(
            needs_layout_passes=False, use_tc_tiling_on_sc=False),
        scratch_types=[
            pltpu.VMEM((2 * _T * _NV,), jnp.float32),
            pltpu.VMEM((_R,), jnp.int32),
            pltpu.VMEM((_R,), jnp.int32),
            pltpu.VMEM((_R, _DP), jnp.float32),
            pltpu.VMEM((_R, _DP), jnp.float32),
            pltpu.SemaphoreType.DMA,
            pltpu.SemaphoreType.DMA,
            pltpu.SemaphoreType.DMA,
            pltpu.SemaphoreType.DMA,
        ],
    )
    return k(xf, tbl)


def kernel(x, tables, weights, biases):
    pad = ((0, 0), (0, _DP - _D))
    w = jnp.pad(jnp.concatenate(weights, axis=0), pad)
    b = jnp.pad(jnp.stack(biases, axis=0), pad)
    num_tables = _make_num_tables(w, b)
    cat_tables = [jnp.pad(t, pad) for t in tables]
    tbl = jnp.concatenate(list(num_tables) + cat_tables, axis=0)
    xf = x.reshape(-1)
    outp = _run(xf, tbl)
    return outp.reshape(_B, _NV, _T, _DP)[..., :_D]


# speedup vs baseline: 2.1817x; 2.1817x over previous
import functools

import jax
import jax.numpy as jnp
from jax import lax
from jax.experimental import pallas as pl
from jax.experimental.pallas import tpu as pltpu
from jax.experimental.pallas import tpu_sc as plsc

_CAT_COUNTS = [4100, 54, 22, 16, 5, 17, 33, 337, 2, 2, 7, 12, 30, 20, 30]
_B = 1024
_T = 50
_D = 50
_DP = 64
_NUMV = 3
_NV = _NUMV + len(_CAT_COUNTS)
_NH = _NV // 2
_VOCAB = 4100

_LIMS = [_VOCAB - 1] * _NUMV + [c - 1 for c in _CAT_COUNTS]

_SMALL_JS = list(range(4, 18))
_SOFF = {}
_srows = 0
for _j in _SMALL_JS:
    _SOFF[_j] = _srows
    _srows += _CAT_COUNTS[_j - 3] + 1
_SROWS = _srows

_NW = 32
_NB = _B // _NW
_XG = 8
_WINDOWS = (0, 16, 32, 34)


def _sc_body(xf_hbm, big_hbm, small_hbm, w_hbm, b_hbm, out_hbm,
             x_v, small_v, w_v, bias_v, idx_v, gbuf, obuf,
             gsem, wsem0, wsem1):
    cid = lax.axis_index("c")
    sid = lax.axis_index("s")
    wid = sid * 2 + cid
    b0 = wid * _NB

    pltpu.sync_copy(small_hbm, small_v)
    pltpu.sync_copy(w_hbm, w_v)
    pltpu.sync_copy(b_hbm, bias_v)

    lanes = lax.iota(jnp.int32, 16)

    wreg = [[w_v[j, pl.ds(off, 16)] for off in _WINDOWS] for j in range(_NUMV)]
    breg = [[bias_v[j, pl.ds(off, 16)] for off in _WINDOWS]
            for j in range(_NUMV)]

    def num_row(xbase, j, t, slot, jj):
        xs = plsc.load_gather(
            x_v, [jnp.full((16,), xbase + t * _NV + j, jnp.int32)])
        for k, off in enumerate(_WINDOWS):
            obuf[slot, jj, t, pl.ds(off, 16)] = xs * wreg[j][k] + breg[j][k]

    def small_row(xbase, j, t, slot, jj):
        xs = plsc.load_gather(
            x_v, [jnp.full((16,), xbase + t * _NV + j, jnp.int32)])
        base = (jnp.minimum(xs.astype(jnp.int32), _LIMS[j]) + _SOFF[j]) * _D
        for off in _WINDOWS:
            obuf[slot, jj, t, pl.ds(off, 16)] = plsc.load_gather(
                small_v, [base + off + lanes])

    def per_b(bb, xbase):
        for off in _WINDOWS:
            t = off + lanes
            vals = plsc.load_gather(x_v, [xbase + t * _NV + 3])
            idx_v[pl.ds(off, 16)] = jnp.minimum(vals.astype(jnp.int32),
                                                _LIMS[3])
        ga = pltpu.async_copy(big_hbm.at[idx_v], gbuf, gsem)

        @pl.when(bb > b0)
        def _():
            pltpu.make_async_copy(obuf.at[0], out_hbm.at[bb, pl.ds(0, _NH)],
                                  wsem0).wait()

        def h1(t, carry):
            for j in range(_NUMV):
                num_row(xbase, j, t, 0, j)
            for j in range(4, _NH):
                small_row(xbase, j, t, 0, j)
            return carry
        lax.fori_loop(0, _T, h1, 0)
        ga.wait()

        def comp(t, carry):
            for off in _WINDOWS:
                obuf[0, 3, t, pl.ds(off, 16)] = gbuf[t, pl.ds(off, 16)]
            return carry
        lax.fori_loop(0, _T, comp, 0)
        pltpu.async_copy(obuf.at[0], out_hbm.at[bb, pl.ds(0, _NH)], wsem0)

        @pl.when(bb > b0)
        def _():
            pltpu.make_async_copy(obuf.at[1], out_hbm.at[bb, pl.ds(_NH, _NH)],
                                  wsem1).wait()

        def h2(t, carry):
            for j in range(_NH, _NV):
                small_row(xbase, j, t, 1, j - _NH)
            return carry
        lax.fori_loop(0, _T, h2, 0)
        pltpu.async_copy(obuf.at[1], out_hbm.at[bb, pl.ds(_NH, _NH)], wsem1)

    def per_octet(i, carry):
        bg = b0 + _XG * i
        pltpu.sync_copy(
            xf_hbm.at[pl.ds(bg * (_T * _NV), _XG * _T * _NV)], x_v)
        for db in range(_XG):
            per_b(bg + db, db * (_T * _NV))
        return carry

    lax.fori_loop(0, _NB // _XG, per_octet, 0)
    pltpu.make_async_copy(obuf.at[0], out_hbm.at[b0, pl.ds(0, _NH)],
                          wsem0).wait()
    pltpu.make_async_copy(obuf.at[1], out_hbm.at[b0, pl.ds(_NH, _NH)],
                          wsem1).wait()


@functools.partial(jax.jit)
def _run(xf, big, small, w, b):
    k = pl.kernel(
        _sc_body,
        out_type=jax.ShapeDtypeStruct((_B, _NV, _T, _D), jnp.float32),
        mesh=plsc.VectorSubcoreMesh(core_axis_name="c", subcore_axis_name="s"),
        compiler_params=pltpu.CompilerParams(
            needs_layout_passes=False, use_tc_tiling_on_sc=False),
        scratch_types=[
            pltpu.VMEM((_XG * _T * _NV,), jnp.float32),
            pltpu.VMEM((_SROWS * _D,), jnp.float32),
            pltpu.VMEM((_NUMV, _D), jnp.float32),
            pltpu.VMEM((_NUMV, _D), jnp.float32),
            pltpu.VMEM((_T,), jnp.int32),
            pltpu.VMEM((_T, _DP), jnp.float32),
            pltpu.VMEM((2, _NH, _T, _D), jnp.float32),
            pltpu.SemaphoreType.DMA,
            pltpu.SemaphoreType.DMA,
            pltpu.SemaphoreType.DMA,
        ],
    )
    return k(xf, big, small, w, b)


def kernel(x, tables, weights, biases):
    w = jnp.concatenate(weights, axis=0)
    b = jnp.stack(biases, axis=0)
    big = jnp.pad(tables[0], ((0, 0), (0, _DP - _D)))
    small = jnp.concatenate(tables[1:], axis=0).reshape(-1)
    xf = x.reshape(-1)
    return _run(xf, big, small, w, b)


# speedup vs baseline: 2.1877x; 1.0028x over previous
import functools

import jax
import jax.numpy as jnp
from jax import lax
from jax.experimental import pallas as pl
from jax.experimental.pallas import tpu as pltpu
from jax.experimental.pallas import tpu_sc as plsc

_CAT_COUNTS = [4100, 54, 22, 16, 5, 17, 33, 337, 2, 2, 7, 12, 30, 20, 30]
_B = 1024
_T = 50
_D = 50
_DP = 64
_NUMV = 3
_NV = _NUMV + len(_CAT_COUNTS)
_NH = _NV // 2
_VOCAB = 4100

_LIMS = [_VOCAB - 1] * _NUMV + [c - 1 for c in _CAT_COUNTS]

_SMALL_JS = list(range(4, 18))
_SOFF = {}
_srows = 0
for _j in _SMALL_JS:
    _SOFF[_j] = _srows
    _srows += _CAT_COUNTS[_j - 3] + 1
_SROWS = _srows

_NW = 32
_NB = _B // _NW
_XG = 8
_WINDOWS = (0, 16, 32, 34)


def _sc_body(xf_hbm, big_hbm, small_hbm, w_hbm, b_hbm, out_hbm,
             x_v, small_v, w_v, bias_v, idx_v, gbuf, obuf,
             gsem, wsem0, wsem1):
    cid = lax.axis_index("c")
    sid = lax.axis_index("s")
    wid = sid * 2 + cid
    b0 = wid * _NB

    pltpu.sync_copy(small_hbm, small_v)
    pltpu.sync_copy(w_hbm, w_v)
    pltpu.sync_copy(b_hbm, bias_v)

    lanes = lax.iota(jnp.int32, 16)

    wreg = [[w_v[j, pl.ds(off, 16)] for off in _WINDOWS] for j in range(_NUMV)]
    breg = [[bias_v[j, pl.ds(off, 16)] for off in _WINDOWS]
            for j in range(_NUMV)]

    def num_row(xbase, j, t, slot, jj):
        xs = plsc.load_gather(
            x_v, [jnp.full((16,), xbase + t * _NV + j, jnp.int32)])
        for k, off in enumerate(_WINDOWS):
            obuf[slot, jj, t, pl.ds(off, 16)] = xs * wreg[j][k] + breg[j][k]

    def small_row(xbase, j, t, slot, jj):
        xs = plsc.load_gather(
            x_v, [jnp.full((16,), xbase + t * _NV + j, jnp.int32)])
        base = (jnp.minimum(xs.astype(jnp.int32), _LIMS[j]) + _SOFF[j]) * _D
        for off in _WINDOWS:
            obuf[slot, jj, t, pl.ds(off, 16)] = plsc.load_gather(
                small_v, [base + off + lanes])

    def per_b(bb, xbase):
        for off in _WINDOWS:
            t = off + lanes
            vals = plsc.load_gather(x_v, [xbase + t * _NV + 3])
            idx_v[pl.ds(off, 16)] = jnp.minimum(vals.astype(jnp.int32),
                                                _LIMS[3])
        ga = pltpu.async_copy(big_hbm.at[idx_v], gbuf, gsem)

        @pl.when(bb > b0)
        def _():
            pltpu.make_async_copy(obuf.at[0], out_hbm.at[bb, pl.ds(0, _NH)],
                                  wsem0).wait()

        def h1(i, carry):
            for u in range(2):
                t = 2 * i + u
                for j in range(_NUMV):
                    num_row(xbase, j, t, 0, j)
                for j in range(4, _NH):
                    small_row(xbase, j, t, 0, j)
            return carry
        lax.fori_loop(0, _T // 2, h1, 0)
        ga.wait()

        def comp(i, carry):
            for u in range(2):
                t = 2 * i + u
                for off in _WINDOWS:
                    obuf[0, 3, t, pl.ds(off, 16)] = gbuf[t, pl.ds(off, 16)]
            return carry
        lax.fori_loop(0, _T // 2, comp, 0)
        pltpu.async_copy(obuf.at[0], out_hbm.at[bb, pl.ds(0, _NH)], wsem0)

        @pl.when(bb > b0)
        def _():
            pltpu.make_async_copy(obuf.at[1], out_hbm.at[bb, pl.ds(_NH, _NH)],
                                  wsem1).wait()

        def h2(i, carry):
            for u in range(2):
                t = 2 * i + u
                for j in range(_NH, _NV):
                    small_row(xbase, j, t, 1, j - _NH)
            return carry
        lax.fori_loop(0, _T // 2, h2, 0)
        pltpu.async_copy(obuf.at[1], out_hbm.at[bb, pl.ds(_NH, _NH)], wsem1)

    def per_octet(i, carry):
        bg = b0 + _XG * i
        pltpu.sync_copy(
            xf_hbm.at[pl.ds(bg * (_T * _NV), _XG * _T * _NV)], x_v)
        for db in range(_XG):
            per_b(bg + db, db * (_T * _NV))
        return carry

    lax.fori_loop(0, _NB // _XG, per_octet, 0)
    pltpu.make_async_copy(obuf.at[0], out_hbm.at[b0, pl.ds(0, _NH)],
                          wsem0).wait()
    pltpu.make_async_copy(obuf.at[1], out_hbm.at[b0, pl.ds(_NH, _NH)],
                          wsem1).wait()


@functools.partial(jax.jit)
def _run(xf, big, small, w, b):
    k = pl.kernel(
        _sc_body,
        out_type=jax.ShapeDtypeStruct((_B, _NV, _T, _D), jnp.float32),
        mesh=plsc.VectorSubcoreMesh(core_axis_name="c", subcore_axis_name="s"),
        compiler_params=pltpu.CompilerParams(
            needs_layout_passes=False, use_tc_tiling_on_sc=False),
        scratch_types=[
            pltpu.VMEM((_XG * _T * _NV,), jnp.float32),
            pltpu.VMEM((_SROWS * _D,), jnp.float32),
            pltpu.VMEM((_NUMV, _D), jnp.float32),
            pltpu.VMEM((_NUMV, _D), jnp.float32),
            pltpu.VMEM((_T,), jnp.int32),
            pltpu.VMEM((_T, _DP), jnp.float32),
            pltpu.VMEM((2, _NH, _T, _D), jnp.float32),
            pltpu.SemaphoreType.DMA,
            pltpu.SemaphoreType.DMA,
            pltpu.SemaphoreType.DMA,
        ],
    )
    return k(xf, big, small, w, b)


def kernel(x, tables, weights, biases):
    w = jnp.concatenate(weights, axis=0)
    b = jnp.stack(biases, axis=0)
    big = jnp.pad(tables[0], ((0, 0), (0, _DP - _D)))
    small = jnp.concatenate(tables[1:], axis=0).reshape(-1)
    xf = x.reshape(-1)
    return _run(xf, big, small, w, b)
